# baseline (speedup 1.0000x reference)
import functools

import jax
import jax.numpy as jnp
from jax import lax
from jax.experimental import pallas as pl
from jax.experimental.pallas import tpu as pltpu
from jax.experimental.pallas import tpu_sc as plsc

N_NODES_C = 100000
NW = 32


_SC_PARAMS = pltpu.CompilerParams(use_tc_tiling_on_sc=False)
_SC_PARAMS_NOLAYOUT = pltpu.CompilerParams(use_tc_tiling_on_sc=False,
                                           needs_layout_passes=False)


def _build_indices(ids, adj2, n_nodes):
    mesh = plsc.VectorSubcoreMesh(core_axis_name="c", subcore_axis_name="s")

    @functools.partial(
        pl.kernel,
        out_type=(
            jax.ShapeDtypeStruct((2048,), jnp.int32),
            jax.ShapeDtypeStruct((20480,), jnp.int32),
            jax.ShapeDtypeStruct((204800,), jnp.int32),
        ),
        mesh=mesh,
        compiler_params=_SC_PARAMS_NOLAYOUT,
        scratch_types=(
            pltpu.VMEM((64,), jnp.int32),
            pltpu.VMEM((64,), jnp.int32),
            pltpu.VMEM((64, 32), jnp.int32),
            pltpu.VMEM((640,), jnp.int32),
            pltpu.VMEM((640, 32), jnp.int32),
            pltpu.VMEM((6400,), jnp.int32),
            pltpu.SemaphoreType.DMA,
        ),
    )
    def k(adj_hbm, ids_hbm, i0_out, i1_out, i2_out,
          sv, i0v, a1, i1v, a2, i2v, sem):
        mp = lax.axis_index("c")
        sblk = lax.axis_index("s")
        moff = mp * n_nodes
        pltpu.sync_copy(ids_hbm.at[pl.ds(sblk * 64, 64)], sv)

        def add_off(i, _):
            o = pl.multiple_of(i * 16, 16)
            i0v[pl.ds(o, 16)] = sv[pl.ds(o, 16)] + moff
            return _

        lax.fori_loop(0, 4, add_off, None)
        pltpu.sync_copy(i0v, i0_out.at[pl.ds(mp * 1024 + sblk * 64, 64)])
        cp = pltpu.make_async_copy(adj_hbm.at[i0v], a1, sem)
        cp.start()
        cp.wait()

        def expand(src_ref, dst_ref, samp, base, n_grp):
            def body(i, _):
                r = i * 16 + lax.iota(jnp.int32, 16)
                c = r * 0 + samp
                v = plsc.load_gather(src_ref, [r, c])
                o = pl.multiple_of(i * 16, 16)
                dst_ref[pl.ds(base + o, 16)] = v + moff
                return _
            lax.fori_loop(0, n_grp, body, None)

        for s1 in range(10):
            expand(a1, i1v, s1, s1 * 64, 4)
        for s1 in range(10):
            pltpu.sync_copy(
                i1v.at[pl.ds(s1 * 64, 64)],
                i1_out.at[pl.ds(mp * 10240 + s1 * 1024 + sblk * 64, 64)])
        for s in range(5):
            pltpu.make_async_copy(
                adj_hbm.at[i1v.at[pl.ds(s * 128, 128)]],
                a2.at[pl.ds(s * 128, 128)], sem).start()
        for s in range(5):
            pltpu.make_async_copy(
                adj_hbm.at[i1v.at[pl.ds(0, 128)]],
                a2.at[pl.ds(0, 128)], sem).wait()
        for s2 in range(10):
            expand(a2, i2v, s2, s2 * 640, 40)
        for s2 in range(10):
            for s1 in range(10):
                pltpu.sync_copy(
                    i2v.at[pl.ds(s2 * 640 + s1 * 64, 64)],
                    i2_out.at[pl.ds(mp * 102400 + s2 * 10240
                                    + s1 * 1024 + sblk * 64, 64)])

    return k(adj2, ids)


def _gather_all(p, idx0, idx1, idx2):
    D = p.shape[1]
    nbuf = 3
    mesh = plsc.VectorSubcoreMesh(core_axis_name="c", subcore_axis_name="s")

    @functools.partial(
        pl.kernel,
        out_type=(
            jax.ShapeDtypeStruct((2048, D), p.dtype),
            jax.ShapeDtypeStruct((20480, D), p.dtype),
            jax.ShapeDtypeStruct((102400, 2 * D), p.dtype),
        ),
        mesh=mesh,
        compiler_params=_SC_PARAMS,
        scratch_types=(
            [pltpu.VMEM((64,), jnp.int32),
             pltpu.VMEM((640,), jnp.int32),
             pltpu.VMEM((6400,), jnp.int32)]
            + [pltpu.VMEM((128, D), p.dtype) for _ in range(nbuf)]
            + [pltpu.SemaphoreType.DMA for _ in range(2 * nbuf)]
        ),
    )
    def k(p_hbm, i0_hbm, i1_hbm, i2_hbm, g0_out, g1_out, g2_out,
          i0v, i1v, i2v, *rest):
        bufs = rest[:nbuf]
        gsem = rest[nbuf:2 * nbuf]
        wsem = rest[2 * nbuf:]
        mp = lax.axis_index("c")
        sblk = lax.axis_index("s")
        pltpu.sync_copy(i0_hbm.at[pl.ds(mp * 1024 + sblk * 64, 64)], i0v)
        for s1 in range(10):
            pltpu.sync_copy(
                i1_hbm.at[pl.ds(mp * 10240 + s1 * 1024 + sblk * 64, 64)],
                i1v.at[pl.ds(s1 * 64, 64)])
        for s2 in range(10):
            pltpu.sync_copy(
                i2_hbm.at[pl.ds(mp * 102400 + s2 * 10240 + sblk * 640, 640)],
                i2v.at[pl.ds(s2 * 640, 640)])

        subs = [(i0v.at[pl.ds(0, 64)],
                 g0_out.at[pl.ds(mp * 1024 + sblk * 64, 64)], 64)]
        subs += [(i1v.at[pl.ds(s1 * 64, 64)],
                  g1_out.at[pl.ds(mp * 10240 + s1 * 1024 + sblk * 64, 64)],
                  64) for s1 in range(10)]
        subs += [(i2v.at[pl.ds(s2 * 640 + c5 * 128, 128)],
                  g2_out.at[pl.ds(mp * 51200 + (s2 // 2) * 10240
                                  + sblk * 640 + c5 * 128, 128),
                            pl.ds((s2 % 2) * D, D)],
                  128) for s2 in range(10) for c5 in range(5)]

        def g_start(j, b):
            iv, _, n = subs[j]
            pltpu.make_async_copy(
                p_hbm.at[iv], bufs[b].at[pl.ds(0, n)], gsem[b]).start()

        def g_wait(j, b):
            n = subs[j][2]
            pltpu.make_async_copy(
                p_hbm.at[subs[0][0]], bufs[b].at[pl.ds(0, n)], gsem[b]).wait()

        def w_start(j, b):
            _, d, n = subs[j]
            pltpu.make_async_copy(bufs[b].at[pl.ds(0, n)], d, wsem[b]).start()

        def w_wait(j, b):
            _, d, n = subs[j]
            pltpu.make_async_copy(bufs[b].at[pl.ds(0, n)], d, wsem[b]).wait()

        n_sub = len(subs)
        for j in range(nbuf):
            g_start(j, j)
        for j in range(n_sub):
            b = j % nbuf
            g_wait(j, b)
            w_start(j, b)
            nxt = j + nbuf
            if nxt < n_sub:
                w_wait(j, b)
                g_start(nxt, b)
        for j in range(max(0, n_sub - nbuf), n_sub):
            w_wait(j, j % nbuf)

    return k(p, idx0, idx1, idx2)


def _proj_body(f_ref, w_ref, o_ref):
    o_ref[0] = jnp.dot(f_ref[...], w_ref[0],
                       preferred_element_type=jnp.float32)


def _project(feats, wcat):
    m = feats.shape[0]
    bm = 5000
    return pl.pallas_call(
        _proj_body,
        grid=(m // bm, 2),
        in_specs=[
            pl.BlockSpec((bm, 128), lambda i, mp: (i, 0)),
            pl.BlockSpec((1, 128, 64), lambda i, mp: (mp, 0, 0)),
        ],
        out_specs=pl.BlockSpec((1, bm, 64), lambda i, mp: (mp, i, 0)),
        out_shape=jax.ShapeDtypeStruct((2, m, 64), jnp.float32),
    )(feats, wcat)


def _onehots():
    r = lax.broadcasted_iota(jnp.int32, (64, 4), 0)
    c = lax.broadcasted_iota(jnp.int32, (64, 4), 1)
    oh = (r // 16 == c).astype(jnp.float32)
    rt = lax.broadcasted_iota(jnp.int32, (4, 64), 0)
    ct = lax.broadcasted_iota(jnp.int32, (4, 64), 1)
    oht = (ct // 16 == rt).astype(jnp.float32)
    return oh, oht


def _head_attn(self64, neigh_list, a_s, a_n):
    oh, oht = _onehots()
    as_row = a_s.reshape(1, 64)
    an_row = a_n.reshape(1, 64)
    dot = functools.partial(jnp.dot, preferred_element_type=jnp.float32)
    ss = dot(self64 * as_row, oh)
    scs = []
    for nb in neigh_list:
        sc = dot(nb * an_row, oh) + ss
        scs.append(jnp.where(sc >= 0, sc, 0.2 * sc))
    mx = scs[0]
    for sc in scs[1:]:
        mx = jnp.maximum(mx, sc)
    es = [jnp.exp(sc - mx) for sc in scs]
    den = es[0]
    for e in es[1:]:
        den = den + e
    agg = None
    for e, nb in zip(es, neigh_list):
        term = dot(e / den, oht) * nb
        agg = term if agg is None else agg + term
    return jnp.maximum(self64 + agg, 0.0)


def _agg_mid_body(g1_ref, g2_ref, as_ref, an_ref, o_ref):
    g2 = g2_ref[0]
    neigh = [g2[s2 // 2][:, (s2 % 2) * 64:(s2 % 2 + 1) * 64]
             for s2 in range(10)]
    o_ref[0] = _head_attn(g1_ref[0], neigh, as_ref[0], an_ref[0])


def _agg_mid(g1, g2p, a0s, a0n):
    c = 2560
    return pl.pallas_call(
        _agg_mid_body,
        grid=(2, 10240 // c),
        in_specs=[
            pl.BlockSpec((1, c, 64), lambda mp, i: (mp, i, 0)),
            pl.BlockSpec((1, 5, c, 128), lambda mp, i: (mp, 0, i, 0)),
            pl.BlockSpec((1, 4, 16), lambda mp, i: (mp, 0, 0)),
            pl.BlockSpec((1, 4, 16), lambda mp, i: (mp, 0, 0)),
        ],
        out_specs=pl.BlockSpec((1, c, 64), lambda mp, i: (mp, i, 0)),
        out_shape=jax.ShapeDtypeStruct((2, 10240, 64), jnp.float32),
    )(g1, g2p, a0s, a0n)


def _agg_top_body(g0_ref, g1_ref, b1_ref, a0s_ref, a0n_ref, w1_ref,
                  a1s_ref, a1n_ref, o_ref):
    neigh0 = [g1_ref[0, s] for s in range(10)]
    b0 = _head_attn(g0_ref[0], neigh0, a0s_ref[0], a0n_ref[0])
    w1cat = jnp.concatenate([w1_ref[0, h] for h in range(4)], axis=1)
    dot = functools.partial(jnp.dot, preferred_element_type=jnp.float32)
    hs = dot(b0, w1cat)
    hn = [dot(b1_ref[0, s], w1cat) for s in range(10)]
    o_ref[0] = _head_attn(hs, hn, a1s_ref[0], a1n_ref[0])


def _agg_top(g0, g1v, b1v, a0s, a0n, w1, a1s, a1n):
    return pl.pallas_call(
        _agg_top_body,
        grid=(2,),
        in_specs=[
            pl.BlockSpec((1, 1024, 64), lambda mp: (mp, 0, 0)),
            pl.BlockSpec((1, 10, 1024, 64), lambda mp: (mp, 0, 0, 0)),
            pl.BlockSpec((1, 10, 1024, 64), lambda mp: (mp, 0, 0, 0)),
            pl.BlockSpec((1, 4, 16), lambda mp: (mp, 0, 0)),
            pl.BlockSpec((1, 4, 16), lambda mp: (mp, 0, 0)),
            pl.BlockSpec((1, 4, 64, 16), lambda mp: (mp, 0, 0, 0)),
            pl.BlockSpec((1, 4, 16), lambda mp: (mp, 0, 0)),
            pl.BlockSpec((1, 4, 16), lambda mp: (mp, 0, 0)),
        ],
        out_specs=pl.BlockSpec((1, 1024, 64), lambda mp: (mp, 0, 0)),
        out_shape=jax.ShapeDtypeStruct((2, 1024, 64), jnp.float32),
    )(g0, g1v, b1v, a0s, a0n, w1, a1s, a1n)


def kernel(ids, feats, adjs, W0, a0_self, a0_neigh, W1, a1_self, a1_neigh):
    n_nodes = feats.shape[0]
    wcat = jnp.transpose(W0, (0, 2, 1, 3)).reshape(2, 128, 64)
    p = _project(feats, wcat).reshape(2 * n_nodes, 64)

    adj2 = adjs.reshape(2 * n_nodes, 32)
    idx0, idx1, idx2 = _build_indices(ids, adj2, n_nodes)
    g0f, g1f, g2f = _gather_all(p, idx0, idx1, idx2)
    g0 = g0f.reshape(2, 1024, 64)
    g1q = g1f.reshape(2, 10240, 64)
    g1v = g1f.reshape(2, 10, 1024, 64)
    g2p = g2f.reshape(2, 5, 10240, 128)

    b1 = _agg_mid(g1q, g2p, a0_self, a0_neigh)
    return _agg_top(g0, g1v, b1.reshape(2, 10, 1024, 64),
                    a0_self, a0_neigh, W1, a1_self, a1_neigh)


# speedup vs baseline: 1.0034x; 1.0034x over previous
import functools

import jax
import jax.numpy as jnp
from jax import lax
from jax.experimental import pallas as pl
from jax.experimental.pallas import tpu as pltpu
from jax.experimental.pallas import tpu_sc as plsc

N_NODES_C = 100000
NW = 32


_SC_PARAMS = pltpu.CompilerParams(use_tc_tiling_on_sc=False)
_SC_PARAMS_NOLAYOUT = pltpu.CompilerParams(use_tc_tiling_on_sc=False,
                                           needs_layout_passes=False)


def _build_indices(ids, adj2, n_nodes):
    mesh = plsc.VectorSubcoreMesh(core_axis_name="c", subcore_axis_name="s")

    @functools.partial(
        pl.kernel,
        out_type=(
            jax.ShapeDtypeStruct((2048,), jnp.int32),
            jax.ShapeDtypeStruct((20480,), jnp.int32),
            jax.ShapeDtypeStruct((204800,), jnp.int32),
        ),
        mesh=mesh,
        compiler_params=_SC_PARAMS_NOLAYOUT,
        scratch_types=(
            pltpu.VMEM((64,), jnp.int32),
            pltpu.VMEM((64,), jnp.int32),
            pltpu.VMEM((64, 32), jnp.int32),
            pltpu.VMEM((640,), jnp.int32),
            pltpu.VMEM((640, 32), jnp.int32),
            pltpu.VMEM((6400,), jnp.int32),
            pltpu.SemaphoreType.DMA,
        ),
    )
    def k(adj_hbm, ids_hbm, i0_out, i1_out, i2_out,
          sv, i0v, a1, i1v, a2, i2v, sem):
        mp = lax.axis_index("c")
        sblk = lax.axis_index("s")
        moff = mp * n_nodes
        pltpu.sync_copy(ids_hbm.at[pl.ds(sblk * 64, 64)], sv)

        def add_off(i, _):
            o = pl.multiple_of(i * 16, 16)
            i0v[pl.ds(o, 16)] = sv[pl.ds(o, 16)] + moff
            return _

        lax.fori_loop(0, 4, add_off, None)
        pltpu.sync_copy(i0v, i0_out.at[pl.ds(mp * 1024 + sblk * 64, 64)])
        cp = pltpu.make_async_copy(adj_hbm.at[i0v], a1, sem)
        cp.start()
        cp.wait()

        def expand(src_ref, dst_ref, samp, base, n_grp):
            def body(i, _):
                r = i * 16 + lax.iota(jnp.int32, 16)
                c = r * 0 + samp
                v = plsc.load_gather(src_ref, [r, c])
                o = pl.multiple_of(i * 16, 16)
                dst_ref[pl.ds(base + o, 16)] = v + moff
                return _
            lax.fori_loop(0, n_grp, body, None)

        for s1 in range(10):
            expand(a1, i1v, s1, s1 * 64, 4)
        for s1 in range(10):
            pltpu.sync_copy(
                i1v.at[pl.ds(s1 * 64, 64)],
                i1_out.at[pl.ds(mp * 10240 + s1 * 1024 + sblk * 64, 64)])
        for s in range(5):
            pltpu.make_async_copy(
                adj_hbm.at[i1v.at[pl.ds(s * 128, 128)]],
                a2.at[pl.ds(s * 128, 128)], sem).start()
        for s in range(5):
            pltpu.make_async_copy(
                adj_hbm.at[i1v.at[pl.ds(0, 128)]],
                a2.at[pl.ds(0, 128)], sem).wait()
        for s2 in range(10):
            expand(a2, i2v, s2, s2 * 640, 40)
        for s2 in range(10):
            for s1 in range(10):
                pltpu.sync_copy(
                    i2v.at[pl.ds(s2 * 640 + s1 * 64, 64)],
                    i2_out.at[pl.ds(mp * 102400 + s2 * 10240
                                    + s1 * 1024 + sblk * 64, 64)])

    return k(adj2, ids)


def _gather_all(p, idx0, idx1, idx2):
    D = p.shape[1]
    nbuf = 3
    mesh = plsc.VectorSubcoreMesh(core_axis_name="c", subcore_axis_name="s")

    @functools.partial(
        pl.kernel,
        out_type=(
            jax.ShapeDtypeStruct((2048, D), p.dtype),
            jax.ShapeDtypeStruct((20480, D), p.dtype),
            jax.ShapeDtypeStruct((102400, 2 * D), p.dtype),
        ),
        mesh=mesh,
        compiler_params=_SC_PARAMS,
        scratch_types=(
            [pltpu.VMEM((64,), jnp.int32),
             pltpu.VMEM((640,), jnp.int32),
             pltpu.VMEM((6400,), jnp.int32)]
            + [pltpu.VMEM((128, D), p.dtype) for _ in range(nbuf)]
            + [pltpu.SemaphoreType.DMA for _ in range(2 * nbuf)]
        ),
    )
    def k(p_hbm, i0_hbm, i1_hbm, i2_hbm, g0_out, g1_out, g2_out,
          i0v, i1v, i2v, *rest):
        bufs = rest[:nbuf]
        gsem = rest[nbuf:2 * nbuf]
        wsem = rest[2 * nbuf:]
        mp = lax.axis_index("c")
        sblk = lax.axis_index("s")
        pltpu.sync_copy(i0_hbm.at[pl.ds(mp * 1024 + sblk * 64, 64)], i0v)
        for s1 in range(10):
            pltpu.sync_copy(
                i1_hbm.at[pl.ds(mp * 10240 + s1 * 1024 + sblk * 64, 64)],
                i1v.at[pl.ds(s1 * 64, 64)])
        for s2 in range(10):
            pltpu.sync_copy(
                i2_hbm.at[pl.ds(mp * 102400 + s2 * 10240 + sblk * 640, 640)],
                i2v.at[pl.ds(s2 * 640, 640)])

        subs = [(i0v.at[pl.ds(0, 64)],
                 g0_out.at[pl.ds(mp * 1024 + sblk * 64, 64)], 64)]
        subs += [(i1v.at[pl.ds(s1 * 64, 64)],
                  g1_out.at[pl.ds(mp * 10240 + s1 * 1024 + sblk * 64, 64)],
                  64) for s1 in range(10)]
        subs += [(i2v.at[pl.ds(s2 * 640 + c5 * 128, 128)],
                  g2_out.at[pl.ds(mp * 51200 + (s2 // 2) * 10240
                                  + sblk * 640 + c5 * 128, 128),
                            pl.ds((s2 % 2) * D, D)],
                  128) for s2 in range(10) for c5 in range(5)]

        def g_start(j, b):
            iv, _, n = subs[j]
            pltpu.make_async_copy(
                p_hbm.at[iv], bufs[b].at[pl.ds(0, n)], gsem[b]).start()

        def g_wait(j, b):
            n = subs[j][2]
            pltpu.make_async_copy(
                p_hbm.at[subs[0][0]], bufs[b].at[pl.ds(0, n)], gsem[b]).wait()

        def w_start(j, b):
            _, d, n = subs[j]
            pltpu.make_async_copy(bufs[b].at[pl.ds(0, n)], d, wsem[b]).start()

        def w_wait(j, b):
            _, d, n = subs[j]
            pltpu.make_async_copy(bufs[b].at[pl.ds(0, n)], d, wsem[b]).wait()

        n_sub = len(subs)
        for j in range(nbuf):
            g_start(j, j)
        for j in range(n_sub):
            b = j % nbuf
            g_wait(j, b)
            w_start(j, b)
            nxt = j + nbuf
            if nxt < n_sub:
                w_wait(j, b)
                g_start(nxt, b)
        for j in range(max(0, n_sub - nbuf), n_sub):
            w_wait(j, j % nbuf)

    return k(p, idx0, idx1, idx2)


def _proj_body(f_ref, w_ref, o_ref):
    o_ref[0] = jnp.dot(f_ref[...], w_ref[0],
                       preferred_element_type=jnp.float32)


def _project(feats, wcat):
    m = feats.shape[0]
    bm = 4000
    return pl.pallas_call(
        _proj_body,
        grid=(m // bm, 2),
        in_specs=[
            pl.BlockSpec((bm, 128), lambda i, mp: (i, 0)),
            pl.BlockSpec((1, 128, 64), lambda i, mp: (mp, 0, 0)),
        ],
        out_specs=pl.BlockSpec((1, bm, 64), lambda i, mp: (mp, i, 0)),
        out_shape=jax.ShapeDtypeStruct((2, m, 64), jnp.float32),
    )(feats, wcat)


def _onehots():
    r = lax.broadcasted_iota(jnp.int32, (64, 4), 0)
    c = lax.broadcasted_iota(jnp.int32, (64, 4), 1)
    oh = (r // 16 == c).astype(jnp.float32)
    rt = lax.broadcasted_iota(jnp.int32, (4, 64), 0)
    ct = lax.broadcasted_iota(jnp.int32, (4, 64), 1)
    oht = (ct // 16 == rt).astype(jnp.float32)
    return oh, oht


def _head_attn(self64, neigh_list, a_s, a_n):
    oh, oht = _onehots()
    as_row = a_s.reshape(1, 64)
    an_row = a_n.reshape(1, 64)
    dot = functools.partial(jnp.dot, preferred_element_type=jnp.float32)
    ss = dot(self64 * as_row, oh)
    scs = []
    for nb in neigh_list:
        sc = dot(nb * an_row, oh) + ss
        scs.append(jnp.where(sc >= 0, sc, 0.2 * sc))
    mx = scs[0]
    for sc in scs[1:]:
        mx = jnp.maximum(mx, sc)
    es = [jnp.exp(sc - mx) for sc in scs]
    den = es[0]
    for e in es[1:]:
        den = den + e
    agg = None
    for e, nb in zip(es, neigh_list):
        term = dot(e / den, oht) * nb
        agg = term if agg is None else agg + term
    return jnp.maximum(self64 + agg, 0.0)


def _agg_mid_body(g1_ref, g2_ref, as_ref, an_ref, o_ref):
    g2 = g2_ref[0]
    neigh = [g2[s2 // 2][:, (s2 % 2) * 64:(s2 % 2 + 1) * 64]
             for s2 in range(10)]
    o_ref[0] = _head_attn(g1_ref[0], neigh, as_ref[0], an_ref[0])


def _agg_mid(g1, g2p, a0s, a0n):
    c = 2048
    return pl.pallas_call(
        _agg_mid_body,
        grid=(2, 10240 // c),
        in_specs=[
            pl.BlockSpec((1, c, 64), lambda mp, i: (mp, i, 0)),
            pl.BlockSpec((1, 5, c, 128), lambda mp, i: (mp, 0, i, 0)),
            pl.BlockSpec((1, 4, 16), lambda mp, i: (mp, 0, 0)),
            pl.BlockSpec((1, 4, 16), lambda mp, i: (mp, 0, 0)),
        ],
        out_specs=pl.BlockSpec((1, c, 64), lambda mp, i: (mp, i, 0)),
        out_shape=jax.ShapeDtypeStruct((2, 10240, 64), jnp.float32),
    )(g1, g2p, a0s, a0n)


def _agg_top_body(g0_ref, g1_ref, b1_ref, a0s_ref, a0n_ref, w1_ref,
                  a1s_ref, a1n_ref, o_ref):
    neigh0 = [g1_ref[0, s] for s in range(10)]
    b0 = _head_attn(g0_ref[0], neigh0, a0s_ref[0], a0n_ref[0])
    w1cat = jnp.concatenate([w1_ref[0, h] for h in range(4)], axis=1)
    dot = functools.partial(jnp.dot, preferred_element_type=jnp.float32)
    hs = dot(b0, w1cat)
    hn = [dot(b1_ref[0, s], w1cat) for s in range(10)]
    o_ref[0] = _head_attn(hs, hn, a1s_ref[0], a1n_ref[0])


def _agg_top(g0, g1v, b1v, a0s, a0n, w1, a1s, a1n):
    return pl.pallas_call(
        _agg_top_body,
        grid=(2,),
        in_specs=[
            pl.BlockSpec((1, 1024, 64), lambda mp: (mp, 0, 0)),
            pl.BlockSpec((1, 10, 1024, 64), lambda mp: (mp, 0, 0, 0)),
            pl.BlockSpec((1, 10, 1024, 64), lambda mp: (mp, 0, 0, 0)),
            pl.BlockSpec((1, 4, 16), lambda mp: (mp, 0, 0)),
            pl.BlockSpec((1, 4, 16), lambda mp: (mp, 0, 0)),
            pl.BlockSpec((1, 4, 64, 16), lambda mp: (mp, 0, 0, 0)),
            pl.BlockSpec((1, 4, 16), lambda mp: (mp, 0, 0)),
            pl.BlockSpec((1, 4, 16), lambda mp: (mp, 0, 0)),
        ],
        out_specs=pl.BlockSpec((1, 1024, 64), lambda mp: (mp, 0, 0)),
        out_shape=jax.ShapeDtypeStruct((2, 1024, 64), jnp.float32),
    )(g0, g1v, b1v, a0s, a0n, w1, a1s, a1n)


def kernel(ids, feats, adjs, W0, a0_self, a0_neigh, W1, a1_self, a1_neigh):
    n_nodes = feats.shape[0]
    wcat = jnp.transpose(W0, (0, 2, 1, 3)).reshape(2, 128, 64)
    p = _project(feats, wcat).reshape(2 * n_nodes, 64)

    adj2 = adjs.reshape(2 * n_nodes, 32)
    idx0, idx1, idx2 = _build_indices(ids, adj2, n_nodes)
    g0f, g1f, g2f = _gather_all(p, idx0, idx1, idx2)
    g0 = g0f.reshape(2, 1024, 64)
    g1q = g1f.reshape(2, 10240, 64)
    g1v = g1f.reshape(2, 10, 1024, 64)
    g2p = g2f.reshape(2, 5, 10240, 128)

    b1 = _agg_mid(g1q, g2p, a0_self, a0_neigh)
    return _agg_top(g0, g1v, b1.reshape(2, 10, 1024, 64),
                    a0_self, a0_neigh, W1, a1_self, a1_neigh)
